# baseline (speedup 1.0000x reference)
import functools

import jax
import jax.numpy as jnp
from jax import lax
from jax.experimental import pallas as pl
from jax.experimental.pallas import tpu as pltpu
from jax.experimental.pallas import tpu_sc as plsc

NUM_CORES = 2
NUM_SUBCORES = 16
NUM_WORKERS = NUM_CORES * NUM_SUBCORES


def _round_up(a, b):
    return (a + b - 1) // b * b


CHUNK = 128


def _make_deg_kernel(n_pad, nch):
    ept = n_pad // NUM_SUBCORES
    mesh = plsc.VectorSubcoreMesh(core_axis_name="c", subcore_axis_name="s")

    @functools.partial(
        pl.kernel,
        mesh=mesh,
        out_type=[jax.ShapeDtypeStruct((n_pad,), jnp.float32),
                  jax.ShapeDtypeStruct((n_pad,), jnp.float32)],
        scratch_types=[
            pltpu.VMEM((nch, CHUNK), jnp.int32),
            pltpu.VMEM((CHUNK,), jnp.float32),
            pltpu.VMEM((ept,), jnp.float32),
            pltpu.VMEM_SHARED((n_pad,), jnp.float32),
        ],
    )
    def deg_kernel(dstp_hbm, out0_hbm, out1_hbm, dst_v, ones_v, zbuf, deg_sh):
        c = lax.axis_index("c")
        s = lax.axis_index("s")
        wid = c * NUM_SUBCORES + s
        pltpu.sync_copy(dstp_hbm.at[wid], dst_v)

        zeros16 = jnp.zeros((16,), jnp.float32)
        ones16 = jnp.ones((16,), jnp.float32)
        for i in range(CHUNK // 16):
            ones_v[pl.ds(i * 16, 16)] = ones16

        def zero_body(k, _):
            zbuf[pl.ds(k * 16, 16)] = zeros16
            return 0

        lax.fori_loop(0, ept // 16, zero_body, 0)
        pltpu.sync_copy(zbuf, deg_sh.at[pl.ds(s * ept, ept)])
        plsc.subcore_barrier()

        def scat_body(j, _):
            pltpu.sync_copy(ones_v, deg_sh.at[dst_v.at[j]], add=True)
            return 0

        lax.fori_loop(0, nch, scat_body, 0)
        plsc.subcore_barrier()

        @pl.when(c == 0)
        def _():
            pltpu.sync_copy(deg_sh.at[pl.ds(s * ept, ept)],
                            out0_hbm.at[pl.ds(s * ept, ept)])

        @pl.when(c == 1)
        def _():
            pltpu.sync_copy(deg_sh.at[pl.ds(s * ept, ept)],
                            out1_hbm.at[pl.ds(s * ept, ept)])

    return deg_kernel


def _matmul_body(x_ref, w1_ref, deg0_ref, deg1_ref, h_ref, dinv_ref):
    deg = deg0_ref[...] + deg1_ref[...] + 1.0
    dinv = lax.rsqrt(deg)
    h = jnp.dot(x_ref[...], w1_ref[...], preferred_element_type=jnp.float32)
    h_ref[...] = h * dinv
    dinv_ref[...] = dinv


def _run_matmul(x, W1, deg0, deg1, n_pad, blk=2000):
    g = x.shape[0] // blk
    return pl.pallas_call(
        _matmul_body,
        grid=(g,),
        in_specs=[
            pl.BlockSpec((blk, 128), lambda i: (i, 0)),
            pl.BlockSpec((128, 128), lambda i: (0, 0)),
            pl.BlockSpec((blk, 1), lambda i: (i, 0)),
            pl.BlockSpec((blk, 1), lambda i: (i, 0)),
        ],
        out_specs=[
            pl.BlockSpec((blk, 128), lambda i: (i, 0)),
            pl.BlockSpec((blk, 1), lambda i: (i, 0)),
        ],
        out_shape=[
            jax.ShapeDtypeStruct((n_pad, 128), jnp.float32),
            jax.ShapeDtypeStruct((n_pad, 1), jnp.float32),
        ],
    )(x, W1, deg0, deg1)


def _make_edge_kernel(n_pad, cpt):
    rpt = n_pad // NUM_SUBCORES
    ns_c = cpt // 8
    mesh = plsc.VectorSubcoreMesh(core_axis_name="c", subcore_axis_name="s")

    @functools.partial(
        pl.kernel,
        mesh=mesh,
        out_type=jax.ShapeDtypeStruct((NUM_CORES, n_pad, 128), jnp.float32),
        scratch_types=[
            pltpu.VMEM((8, CHUNK), jnp.int32),
            pltpu.VMEM((8, CHUNK), jnp.int32),
            pltpu.VMEM((8, CHUNK), jnp.int32),
            pltpu.VMEM((8, CHUNK), jnp.int32),
            pltpu.VMEM((CHUNK, 128), jnp.float32),
            pltpu.VMEM((CHUNK, 128), jnp.float32),
            pltpu.VMEM_SHARED((n_pad, 128), jnp.float32),
            pltpu.SemaphoreType.DMA,
            pltpu.SemaphoreType.DMA,
            pltpu.SemaphoreType.DMA,
        ],
    )
    def edge_kernel(h_hbm, srcp_hbm, dstp_hbm, zrow_hbm, out_hbm,
                    sidx_a, didx_a, sidx_b, didx_b, buf0, buf1, acc_sh,
                    gsem0, gsem1, isem):
        c = lax.axis_index("c")
        s = lax.axis_index("s")
        wid = c * NUM_SUBCORES + s
        rs = s * rpt
        off = wid * cpt
        ns = ns_c

        pltpu.sync_copy(zrow_hbm, acc_sh.at[pl.ds(rs, rpt)])

        pltpu.sync_copy(srcp_hbm.at[pl.ds(off, 8)], sidx_a)
        pltpu.sync_copy(dstp_hbm.at[pl.ds(off, 8)], didx_a)
        plsc.subcore_barrier()
        bufs = (buf0, buf1)
        gsems = (gsem0, gsem1)
        sidxs = (sidx_a, sidx_b)
        didxs = (didx_a, didx_b)
        pltpu.async_copy(h_hbm.at[sidx_a.at[0]], buf0, gsem0)

        def stripe_pair(i, _):
            for p in range(2):
                t = i * 2 + p
                sidx, didx = sidxs[p], didxs[p]
                nsidx, ndidx = sidxs[1 - p], didxs[1 - p]
                nbase = off + (t + 1) * 8

                @pl.when(t + 1 < ns)
                def _():
                    pltpu.async_copy(srcp_hbm.at[pl.ds(nbase, 8)], nsidx,
                                     isem)
                    pltpu.async_copy(dstp_hbm.at[pl.ds(nbase, 8)], ndidx,
                                     isem)

                for cc in range(8):
                    if cc < 7:
                        pltpu.async_copy(h_hbm.at[sidx.at[cc + 1]],
                                         bufs[(cc + 1) % 2],
                                         gsems[(cc + 1) % 2])
                    else:
                        @pl.when(t + 1 < ns)
                        def _():
                            pltpu.make_async_copy(
                                srcp_hbm.at[pl.ds(nbase, 8)], nsidx,
                                isem).wait()
                            pltpu.make_async_copy(
                                dstp_hbm.at[pl.ds(nbase, 8)], ndidx,
                                isem).wait()
                            pltpu.async_copy(h_hbm.at[nsidx.at[0]],
                                             bufs[0], gsems[0])
                    pltpu.make_async_copy(h_hbm.at[sidx.at[cc]],
                                          bufs[cc % 2], gsems[cc % 2]).wait()
                    pltpu.sync_copy(bufs[cc % 2], acc_sh.at[didx.at[cc]],
                                    add=True)
            return 0

        lax.fori_loop(0, ns // 2, stripe_pair, 0)
        plsc.subcore_barrier()
        pltpu.sync_copy(acc_sh.at[pl.ds(rs, rpt)],
                        out_hbm.at[c, pl.ds(rs, rpt)])

    return edge_kernel


def _make_pool_body(num_blocks, num_graphs, num_classes):
    def pool_body(acc_ref, h_ref, dinv_ref, b1_ref, batch_ref, w2_ref,
                  b2_ref, out_ref, pool_sc):
        j = pl.program_id(0)

        @pl.when(j == 0)
        def _():
            pool_sc[...] = jnp.full_like(pool_sc[...], -jnp.inf)

        a = acc_ref[0] + acc_ref[1] + h_ref[...]
        z = jnp.maximum(a * dinv_ref[...] + b1_ref[...], 0.0)
        batch = batch_ref[...]

        def seg_body(g, _):
            m = jnp.where(batch == g, z, -jnp.inf)
            m = jnp.max(m, axis=0, keepdims=True)
            cur = pool_sc[pl.ds(g, 1), :]
            pool_sc[pl.ds(g, 1), :] = jnp.maximum(cur, m)
            return 0

        glo = batch[0, 0]
        ghi = jnp.minimum(batch[batch.shape[0] - 1, 0], num_graphs - 1)
        lax.fori_loop(glo, ghi + 1, seg_body, 0)

        @pl.when(j == num_blocks - 1)
        def _():
            pooled = pool_sc[...]
            logits = jnp.dot(pooled, w2_ref[...],
                             preferred_element_type=jnp.float32) + b2_ref[...]
            cols = lax.broadcasted_iota(jnp.int32, logits.shape, 1)
            valid = cols < num_classes
            lm = jnp.where(valid, logits, -jnp.inf)
            mx = jnp.max(lm, axis=-1, keepdims=True)
            e = jnp.where(valid, jnp.exp(lm - mx), 0.0)
            ssum = jnp.sum(e, axis=-1, keepdims=True)
            out_ref[...] = lm - mx - jnp.log(ssum)

    return pool_body


def _run_pool(acc, h_scaled, dinv, b1r, batch_p, W2p, b2p, n_pad,
              num_graphs, num_classes, blk=1024):
    g = n_pad // blk
    return pl.pallas_call(
        _make_pool_body(g, num_graphs, num_classes),
        grid=(g,),
        in_specs=[
            pl.BlockSpec((NUM_CORES, blk, 128), lambda i: (0, i, 0)),
            pl.BlockSpec((blk, 128), lambda i: (i, 0)),
            pl.BlockSpec((blk, 1), lambda i: (i, 0)),
            pl.BlockSpec((1, 128), lambda i: (0, 0)),
            pl.BlockSpec((blk, 1), lambda i: (i, 0)),
            pl.BlockSpec((128, 128), lambda i: (0, 0)),
            pl.BlockSpec((1, 128), lambda i: (0, 0)),
        ],
        out_specs=pl.BlockSpec((num_graphs, 128), lambda i: (0, 0)),
        out_shape=jax.ShapeDtypeStruct((num_graphs, 128), jnp.float32),
        scratch_shapes=[pltpu.VMEM((num_graphs, 128), jnp.float32)],
    )(acc, h_scaled, dinv, b1r, batch_p, W2p, b2p)


def kernel(x, edge_index, batch, W1, b1, W2, b2):
    n, d = x.shape
    e = edge_index.shape[1]
    num_graphs = 64
    num_classes = W2.shape[1]

    n_pad = _round_up(n + 1, 2560)
    cpt = _round_up(-(-e // (NUM_WORKERS * CHUNK)), 16)
    tch = NUM_WORKERS * cpt
    e_pad = tch * CHUNK

    src = edge_index[0].astype(jnp.int32)
    dst = edge_index[1].astype(jnp.int32)
    pad = e_pad - e
    src_p = jnp.concatenate([src, jnp.full((pad,), n, jnp.int32)])
    dst_p = jnp.concatenate([dst, jnp.full((pad,), n, jnp.int32)])
    src_p = src_p.reshape(tch, CHUNK)
    dst_p = dst_p.reshape(tch, CHUNK)

    batch_p = jnp.full((n_pad, 1), num_graphs, jnp.int32).at[:n, 0].set(
        batch.astype(jnp.int32))
    W2p = jnp.zeros((d, 128), jnp.float32).at[:, :num_classes].set(W2)
    b2p = jnp.zeros((1, 128), jnp.float32).at[0, :num_classes].set(b2)
    b1r = b1.reshape(1, d)

    deg0, deg1 = _make_deg_kernel(n_pad, cpt)(
        dst_p.reshape(NUM_WORKERS, cpt, CHUNK))
    h_scaled, dinv = _run_matmul(x, W1, deg0.reshape(n_pad, 1),
                                 deg1.reshape(n_pad, 1), n_pad)
    zrow = jnp.zeros((n_pad // NUM_SUBCORES, 128), jnp.float32)
    acc = _make_edge_kernel(n_pad, cpt)(h_scaled, src_p, dst_p, zrow)
    out128 = _run_pool(acc, h_scaled, dinv, b1r, batch_p, W2p, b2p, n_pad,
                       num_graphs, num_classes)
    return out128[:, :num_classes]


# speedup vs baseline: 1.0662x; 1.0662x over previous
import functools

import jax
import jax.numpy as jnp
from jax import lax
from jax.experimental import pallas as pl
from jax.experimental.pallas import tpu as pltpu
from jax.experimental.pallas import tpu_sc as plsc

NUM_CORES = 2
NUM_SUBCORES = 16
NUM_WORKERS = NUM_CORES * NUM_SUBCORES


def _round_up(a, b):
    return (a + b - 1) // b * b


CHUNK = 128


def _make_deg_kernel(n_pad, nch):
    ept = n_pad // NUM_SUBCORES
    mesh = plsc.VectorSubcoreMesh(core_axis_name="c", subcore_axis_name="s")

    @functools.partial(
        pl.kernel,
        mesh=mesh,
        out_type=[jax.ShapeDtypeStruct((n_pad,), jnp.float32),
                  jax.ShapeDtypeStruct((n_pad,), jnp.float32)],
        scratch_types=[
            pltpu.VMEM((nch, CHUNK), jnp.int32),
            pltpu.VMEM((CHUNK,), jnp.float32),
            pltpu.VMEM((ept,), jnp.float32),
            pltpu.VMEM_SHARED((n_pad,), jnp.float32),
        ],
    )
    def deg_kernel(dstp_hbm, out0_hbm, out1_hbm, dst_v, ones_v, zbuf, deg_sh):
        c = lax.axis_index("c")
        s = lax.axis_index("s")
        wid = c * NUM_SUBCORES + s
        pltpu.sync_copy(dstp_hbm.at[wid], dst_v)

        zeros16 = jnp.zeros((16,), jnp.float32)
        ones16 = jnp.ones((16,), jnp.float32)
        for i in range(CHUNK // 16):
            ones_v[pl.ds(i * 16, 16)] = ones16

        def zero_body(k, _):
            zbuf[pl.ds(k * 16, 16)] = zeros16
            return 0

        lax.fori_loop(0, ept // 16, zero_body, 0)
        pltpu.sync_copy(zbuf, deg_sh.at[pl.ds(s * ept, ept)])
        plsc.subcore_barrier()

        def scat_body(j, _):
            pltpu.sync_copy(ones_v, deg_sh.at[dst_v.at[j]], add=True)
            return 0

        lax.fori_loop(0, nch, scat_body, 0)
        plsc.subcore_barrier()

        @pl.when(c == 0)
        def _():
            pltpu.sync_copy(deg_sh.at[pl.ds(s * ept, ept)],
                            out0_hbm.at[pl.ds(s * ept, ept)])

        @pl.when(c == 1)
        def _():
            pltpu.sync_copy(deg_sh.at[pl.ds(s * ept, ept)],
                            out1_hbm.at[pl.ds(s * ept, ept)])

    return deg_kernel


def _matmul_body(x_ref, w1_ref, deg0_ref, deg1_ref, h_ref, dinv_ref):
    deg = deg0_ref[...] + deg1_ref[...] + 1.0
    dinv = lax.rsqrt(deg)
    h = jnp.dot(x_ref[...], w1_ref[...], preferred_element_type=jnp.float32)
    h_ref[...] = h * dinv
    dinv_ref[...] = dinv


def _run_matmul(x, W1, deg0, deg1, n_pad, blk=2000):
    g = x.shape[0] // blk
    return pl.pallas_call(
        _matmul_body,
        grid=(g,),
        in_specs=[
            pl.BlockSpec((blk, 128), lambda i: (i, 0)),
            pl.BlockSpec((128, 128), lambda i: (0, 0)),
            pl.BlockSpec((blk, 1), lambda i: (i, 0)),
            pl.BlockSpec((blk, 1), lambda i: (i, 0)),
        ],
        out_specs=[
            pl.BlockSpec((blk, 128), lambda i: (i, 0)),
            pl.BlockSpec((blk, 1), lambda i: (i, 0)),
        ],
        out_shape=[
            jax.ShapeDtypeStruct((n_pad, 128), jnp.float32),
            jax.ShapeDtypeStruct((n_pad, 1), jnp.float32),
        ],
    )(x, W1, deg0, deg1)


ECH = 80


def _make_edge_kernel(n_pad, cpt):
    rpt = n_pad // NUM_SUBCORES
    ns_c = cpt // 8
    mesh = plsc.VectorSubcoreMesh(core_axis_name="c", subcore_axis_name="s")

    @functools.partial(
        pl.kernel,
        mesh=mesh,
        out_type=jax.ShapeDtypeStruct((NUM_CORES, n_pad, 128), jnp.float32),
        scratch_types=[
            pltpu.VMEM((8, ECH), jnp.int32),
            pltpu.VMEM((8, ECH), jnp.int32),
            pltpu.VMEM((8, ECH), jnp.int32),
            pltpu.VMEM((8, ECH), jnp.int32),
            pltpu.VMEM((ECH, 128), jnp.float32),
            pltpu.VMEM((ECH, 128), jnp.float32),
            pltpu.VMEM((ECH, 128), jnp.float32),
            pltpu.VMEM((ECH, 128), jnp.float32),
            pltpu.VMEM_SHARED((n_pad, 128), jnp.float32),
            pltpu.SemaphoreType.DMA,
            pltpu.SemaphoreType.DMA,
            pltpu.SemaphoreType.DMA,
            pltpu.SemaphoreType.DMA,
            pltpu.SemaphoreType.DMA,
        ],
    )
    def edge_kernel(h_hbm, srcp_hbm, dstp_hbm, zrow_hbm, out_hbm,
                    sidx_a, didx_a, sidx_b, didx_b,
                    buf0, buf1, buf2, buf3, acc_sh,
                    gsem0, gsem1, gsem2, gsem3, isem):
        c = lax.axis_index("c")
        s = lax.axis_index("s")
        wid = c * NUM_SUBCORES + s
        rs = s * rpt
        off = wid * cpt
        ns = ns_c

        pltpu.sync_copy(zrow_hbm, acc_sh.at[pl.ds(rs, rpt)])

        pltpu.sync_copy(srcp_hbm.at[pl.ds(off, 8)], sidx_a)
        pltpu.sync_copy(dstp_hbm.at[pl.ds(off, 8)], didx_a)
        plsc.subcore_barrier()
        bufs = (buf0, buf1, buf2, buf3)
        gsems = (gsem0, gsem1, gsem2, gsem3)
        sidxs = (sidx_a, sidx_b)
        didxs = (didx_a, didx_b)
        for k in range(3):
            pltpu.async_copy(h_hbm.at[sidx_a.at[k]], bufs[k], gsems[k])

        def stripe_pair(i, _):
            for p in range(2):
                t = i * 2 + p
                sidx, didx = sidxs[p], didxs[p]
                nsidx, ndidx = sidxs[1 - p], didxs[1 - p]
                nbase = off + (t + 1) * 8

                @pl.when(t + 1 < ns)
                def _():
                    pltpu.async_copy(srcp_hbm.at[pl.ds(nbase, 8)], nsidx,
                                     isem)
                    pltpu.async_copy(dstp_hbm.at[pl.ds(nbase, 8)], ndidx,
                                     isem)

                for cc in range(8):
                    b = cc % 4
                    pf = (cc + 3) % 4
                    if cc < 5:
                        pltpu.async_copy(h_hbm.at[sidx.at[cc + 3]],
                                         bufs[pf], gsems[pf])
                    elif cc == 5:
                        @pl.when(t + 1 < ns)
                        def _():
                            pltpu.make_async_copy(
                                srcp_hbm.at[pl.ds(nbase, 8)], nsidx,
                                isem).wait()
                            pltpu.make_async_copy(
                                dstp_hbm.at[pl.ds(nbase, 8)], ndidx,
                                isem).wait()
                            pltpu.async_copy(h_hbm.at[nsidx.at[0]],
                                             bufs[pf], gsems[pf])
                    else:
                        @pl.when(t + 1 < ns)
                        def _():
                            pltpu.async_copy(h_hbm.at[nsidx.at[cc - 5]],
                                             bufs[pf], gsems[pf])
                    pltpu.make_async_copy(h_hbm.at[sidx.at[cc]],
                                          bufs[b], gsems[b]).wait()
                    pltpu.sync_copy(bufs[b], acc_sh.at[didx.at[cc]],
                                    add=True)
            return 0

        lax.fori_loop(0, ns // 2, stripe_pair, 0)
        plsc.subcore_barrier()
        pltpu.sync_copy(acc_sh.at[pl.ds(rs, rpt)],
                        out_hbm.at[c, pl.ds(rs, rpt)])

    return edge_kernel


def _make_pool_body(num_blocks, num_graphs, num_classes):
    def pool_body(acc_ref, h_ref, dinv_ref, b1_ref, batch_ref, w2_ref,
                  b2_ref, out_ref, pool_sc):
        j = pl.program_id(0)

        @pl.when(j == 0)
        def _():
            pool_sc[...] = jnp.full_like(pool_sc[...], -jnp.inf)

        a = acc_ref[0] + acc_ref[1] + h_ref[...]
        z = jnp.maximum(a * dinv_ref[...] + b1_ref[...], 0.0)
        batch = batch_ref[...]

        def seg_body(g, _):
            m = jnp.where(batch == g, z, -jnp.inf)
            m = jnp.max(m, axis=0, keepdims=True)
            cur = pool_sc[pl.ds(g, 1), :]
            pool_sc[pl.ds(g, 1), :] = jnp.maximum(cur, m)
            return 0

        glo = batch[0, 0]
        ghi = jnp.minimum(batch[batch.shape[0] - 1, 0], num_graphs - 1)
        lax.fori_loop(glo, ghi + 1, seg_body, 0)

        @pl.when(j == num_blocks - 1)
        def _():
            pooled = pool_sc[...]
            logits = jnp.dot(pooled, w2_ref[...],
                             preferred_element_type=jnp.float32) + b2_ref[...]
            cols = lax.broadcasted_iota(jnp.int32, logits.shape, 1)
            valid = cols < num_classes
            lm = jnp.where(valid, logits, -jnp.inf)
            mx = jnp.max(lm, axis=-1, keepdims=True)
            e = jnp.where(valid, jnp.exp(lm - mx), 0.0)
            ssum = jnp.sum(e, axis=-1, keepdims=True)
            out_ref[...] = lm - mx - jnp.log(ssum)

    return pool_body


def _run_pool(acc, h_scaled, dinv, b1r, batch_p, W2p, b2p, n_pad,
              num_graphs, num_classes, blk=1024):
    g = n_pad // blk
    return pl.pallas_call(
        _make_pool_body(g, num_graphs, num_classes),
        grid=(g,),
        in_specs=[
            pl.BlockSpec((NUM_CORES, blk, 128), lambda i: (0, i, 0)),
            pl.BlockSpec((blk, 128), lambda i: (i, 0)),
            pl.BlockSpec((blk, 1), lambda i: (i, 0)),
            pl.BlockSpec((1, 128), lambda i: (0, 0)),
            pl.BlockSpec((blk, 1), lambda i: (i, 0)),
            pl.BlockSpec((128, 128), lambda i: (0, 0)),
            pl.BlockSpec((1, 128), lambda i: (0, 0)),
        ],
        out_specs=pl.BlockSpec((num_graphs, 128), lambda i: (0, 0)),
        out_shape=jax.ShapeDtypeStruct((num_graphs, 128), jnp.float32),
        scratch_shapes=[pltpu.VMEM((num_graphs, 128), jnp.float32)],
    )(acc, h_scaled, dinv, b1r, batch_p, W2p, b2p)


def kernel(x, edge_index, batch, W1, b1, W2, b2):
    n, d = x.shape
    e = edge_index.shape[1]
    num_graphs = 64
    num_classes = W2.shape[1]

    n_pad = _round_up(n + 1, 2560)
    cpt = _round_up(-(-e // (NUM_WORKERS * ECH)), 16)
    tch = NUM_WORKERS * cpt
    e_pad = tch * ECH
    assert e_pad % (NUM_WORKERS * CHUNK) == 0

    src = edge_index[0].astype(jnp.int32)
    dst = edge_index[1].astype(jnp.int32)
    pad = e_pad - e
    src_p = jnp.concatenate([src, jnp.full((pad,), n, jnp.int32)])
    dst_p = jnp.concatenate([dst, jnp.full((pad,), n, jnp.int32)])
    src_p = src_p.reshape(tch, ECH)
    dst_p = dst_p.reshape(tch, ECH)

    batch_p = jnp.full((n_pad, 1), num_graphs, jnp.int32).at[:n, 0].set(
        batch.astype(jnp.int32))
    W2p = jnp.zeros((d, 128), jnp.float32).at[:, :num_classes].set(W2)
    b2p = jnp.zeros((1, 128), jnp.float32).at[0, :num_classes].set(b2)
    b1r = b1.reshape(1, d)

    deg0, deg1 = _make_deg_kernel(n_pad, e_pad // (NUM_WORKERS * CHUNK))(
        dst_p.reshape(NUM_WORKERS, e_pad // (NUM_WORKERS * CHUNK), CHUNK))
    h_scaled, dinv = _run_matmul(x, W1, deg0.reshape(n_pad, 1),
                                 deg1.reshape(n_pad, 1), n_pad)
    zrow = jnp.zeros((n_pad // NUM_SUBCORES, 128), jnp.float32)
    acc = _make_edge_kernel(n_pad, cpt)(h_scaled, src_p, dst_p, zrow)
    out128 = _run_pool(acc, h_scaled, dinv, b1r, batch_p, W2p, b2p, n_pad,
                       num_graphs, num_classes)
    return out128[:, :num_classes]
